# baseline (speedup 1.0000x reference)
import functools

import jax
import jax.numpy as jnp
from jax.experimental import pallas as pl
import jax.experimental.pallas.tpu as pltpu

_T = 2048
_H = 1024
_I = 2816
_E = 8
_B = 256
_NB = 24


def _router_kernel(x_ref, gw_ref, pos0_ref, pos1_ref, tw0_ref, tw1_ref,
                   be_ref, bv_ref):
    x = x_ref[...]
    gw = gw_ref[...]
    logits = jax.lax.dot_general(
        x, gw, (((1,), (1,)), ((), ())),
        precision=jax.lax.Precision.DEFAULT,
        preferred_element_type=jnp.float32)
    m = jnp.max(logits, axis=1, keepdims=True)
    ex = jnp.exp(logits - m)
    probs = ex / jnp.sum(ex, axis=1, keepdims=True)

    iota_e = jax.lax.broadcasted_iota(jnp.int32, (_T, _E), 1)
    m1 = jnp.max(probs, axis=1, keepdims=True)
    i0 = jnp.min(jnp.where(probs == m1, iota_e, _E), axis=1, keepdims=True)
    a0 = (iota_e == i0)
    probs2 = jnp.where(a0, -1.0, probs)
    m2 = jnp.max(probs2, axis=1, keepdims=True)
    i1 = jnp.min(jnp.where(probs2 == m2, iota_e, _E), axis=1, keepdims=True)
    a1 = (iota_e == i1)

    a0i = a0.astype(jnp.int32)
    a1i = a1.astype(jnp.int32)

    def inclusive_scan(a):
        c = a
        d = 1
        while d < _T:
            c = c + jnp.concatenate(
                [jnp.zeros((d, _E), jnp.int32), c[:-d]], axis=0)
            d *= 2
        return c

    c0 = inclusive_scan(a0i)
    c1 = inclusive_scan(a1i)
    cnt0 = c0[_T - 1:_T, :]
    cnt1 = c1[_T - 1:_T, :]
    counts = cnt0 + cnt1
    aligned = jnp.bitwise_and(counts + (_B - 1), ~(_B - 1))

    offs_cols = []
    run = jnp.zeros((1, 1), jnp.int32)
    for e in range(_E):
        offs_cols.append(run)
        run = run + aligned[:, e:e + 1]
    offs = jnp.concatenate(offs_cols, axis=1)
    total = run

    rank0 = jnp.sum(a0i * (c0 - 1), axis=1, keepdims=True)
    rank1 = jnp.sum(a1i * (cnt0 + c1 - 1), axis=1, keepdims=True)
    off0 = jnp.sum(a0i * offs, axis=1, keepdims=True)
    off1 = jnp.sum(a1i * offs, axis=1, keepdims=True)
    pos0_ref[...] = off0 + rank0
    pos1_ref[...] = off1 + rank1
    tw0_ref[...] = m1
    tw1_ref[...] = m2

    bstart = _B * jax.lax.broadcasted_iota(jnp.int32, (_NB, 1), 0)
    inb = jnp.logical_and(bstart >= offs, bstart < offs + aligned)
    e_row = jax.lax.broadcasted_iota(jnp.int32, (_NB, _E), 1)
    be = jnp.sum(jnp.where(inb, e_row, 0), axis=1, keepdims=True)
    bv = jnp.sum(inb.astype(jnp.int32), axis=1, keepdims=True)
    lastq = total - 1
    in_last = jnp.logical_and(lastq >= offs, lastq < offs + aligned)
    e_last = jnp.sum(jnp.where(
        in_last, jax.lax.broadcasted_iota(jnp.int32, (1, _E), 1), 0),
        axis=1, keepdims=True)
    be_ref[...] = jnp.where(bv > 0, be, e_last)
    bv_ref[...] = bv


def _ffn_kernel(be_ref, bv_ref, pos0_ref, pos1_ref, tw0_ref, tw1_ref,
                x_ref, w1_ref, w3_ref, w2_ref, out_ref):
    c = pl.program_id(0)
    j = pl.program_id(1)
    b = 2 * j + c

    @pl.when(j == 0)
    def _init():
        out_ref[...] = jnp.zeros_like(out_ref)

    @pl.when(bv_ref[b] > 0)
    def _body():
        slots = _B * b + jax.lax.broadcasted_iota(jnp.int32, (_B, 1), 0)
        p0 = pos0_ref[...]
        p1 = pos1_ref[...]
        m0 = (p0 == slots)
        m1 = (p1 == slots)
        g = jnp.logical_or(m0, m1).astype(jnp.bfloat16)
        wslot = jnp.sum(jnp.where(m0, tw0_ref[...], 0.0) +
                        jnp.where(m1, tw1_ref[...], 0.0),
                        axis=1, keepdims=True)
        xg = jax.lax.dot_general(
            g, x_ref[...], (((1,), (0,)), ((), ())),
            preferred_element_type=jnp.float32).astype(jnp.bfloat16)
        w1 = w1_ref[0]
        w3 = w3_ref[0]
        w2 = w2_ref[0]
        gp = jax.lax.dot_general(
            xg, w1, (((1,), (1,)), ((), ())),
            preferred_element_type=jnp.float32)
        up = jax.lax.dot_general(
            xg, w3, (((1,), (1,)), ((), ())),
            preferred_element_type=jnp.float32)
        h = (gp * jax.nn.sigmoid(gp) * up).astype(jnp.bfloat16)
        y = jax.lax.dot_general(
            h, w2, (((1,), (1,)), ((), ())),
            preferred_element_type=jnp.float32)
        yw = (y * wslot).astype(jnp.bfloat16)
        out_ref[...] += jax.lax.dot_general(
            g, yw, (((0,), (0,)), ((), ())),
            preferred_element_type=jnp.float32)


def _sum2_kernel(p_ref, out_ref):
    out_ref[...] = p_ref[0] + p_ref[1]


@jax.jit
def kernel(hidden_states, gate_w, w1, w3, w2):
    f32 = jnp.float32
    router_out = pl.pallas_call(
        _router_kernel,
        out_shape=[
            jax.ShapeDtypeStruct((_T, 1), jnp.int32),
            jax.ShapeDtypeStruct((_T, 1), jnp.int32),
            jax.ShapeDtypeStruct((_T, 1), f32),
            jax.ShapeDtypeStruct((_T, 1), f32),
            jax.ShapeDtypeStruct((_NB, 1), jnp.int32),
            jax.ShapeDtypeStruct((_NB, 1), jnp.int32),
        ],
    )(hidden_states, gate_w)
    pos0, pos1, tw0, tw1, be, bv = router_out
    pos0 = pos0.reshape(1, _T)
    pos1 = pos1.reshape(1, _T)
    tw0 = tw0.reshape(1, _T)
    tw1 = tw1.reshape(1, _T)
    be = be.reshape(_NB)
    bv = bv.reshape(_NB)

    bf16 = jnp.bfloat16
    x_bf = hidden_states.astype(bf16)
    w1b = w1.astype(bf16)
    w3b = w3.astype(bf16)
    w2b = w2.astype(bf16)

    grid_spec = pltpu.PrefetchScalarGridSpec(
        num_scalar_prefetch=2,
        grid=(2, _NB // 2),
        in_specs=[
            pl.BlockSpec((1, _T), lambda c, j, be, bv: (0, 0)),
            pl.BlockSpec((1, _T), lambda c, j, be, bv: (0, 0)),
            pl.BlockSpec((1, _T), lambda c, j, be, bv: (0, 0)),
            pl.BlockSpec((1, _T), lambda c, j, be, bv: (0, 0)),
            pl.BlockSpec((_T, _H), lambda c, j, be, bv: (0, 0)),
            pl.BlockSpec((1, _I, _H),
                         lambda c, j, be, bv: (be[2 * j + c], 0, 0)),
            pl.BlockSpec((1, _I, _H),
                         lambda c, j, be, bv: (be[2 * j + c], 0, 0)),
            pl.BlockSpec((1, _H, _I),
                         lambda c, j, be, bv: (be[2 * j + c], 0, 0)),
        ],
        out_specs=pl.BlockSpec((1, _T, _H), lambda c, j, be, bv: (c, 0, 0)),
    )
    partial = pl.pallas_call(
        _ffn_kernel,
        grid_spec=grid_spec,
        out_shape=jax.ShapeDtypeStruct((2, _T, _H), f32),
        compiler_params=pltpu.CompilerParams(
            dimension_semantics=("parallel", "arbitrary"),
            vmem_limit_bytes=64 * 1024 * 1024),
    )(be, bv, pos0, pos1, tw0, tw1, x_bf, w1b, w3b, w2b)
    out = pl.pallas_call(
        _sum2_kernel,
        out_shape=jax.ShapeDtypeStruct((_T, _H), f32),
    )(partial)
    return out


# speedup vs baseline: 1.1518x; 1.1518x over previous
import functools

import jax
import jax.numpy as jnp
from jax.experimental import pallas as pl
import jax.experimental.pallas.tpu as pltpu

_T = 2048
_H = 1024
_I = 2816
_E = 8
_B = 256
_NB = 24


def _router_kernel(x_ref, gw_ref, pos0_ref, pos1_ref, tw0_ref, tw1_ref,
                   be_ref, bv_ref, xbf_ref):
    x = x_ref[...]
    xbf_ref[...] = x.astype(jnp.bfloat16)
    gw = gw_ref[...]
    logits = jax.lax.dot_general(
        x, gw, (((1,), (1,)), ((), ())),
        precision=jax.lax.Precision.DEFAULT,
        preferred_element_type=jnp.float32)
    m = jnp.max(logits, axis=1, keepdims=True)
    ex = jnp.exp(logits - m)
    probs = ex / jnp.sum(ex, axis=1, keepdims=True)

    iota_e = jax.lax.broadcasted_iota(jnp.int32, (_T, _E), 1)
    m1 = jnp.max(probs, axis=1, keepdims=True)
    i0 = jnp.min(jnp.where(probs == m1, iota_e, _E), axis=1, keepdims=True)
    a0 = (iota_e == i0)
    probs2 = jnp.where(a0, -1.0, probs)
    m2 = jnp.max(probs2, axis=1, keepdims=True)
    i1 = jnp.min(jnp.where(probs2 == m2, iota_e, _E), axis=1, keepdims=True)
    a1 = (iota_e == i1)

    a0i = a0.astype(jnp.int32)
    a1i = a1.astype(jnp.int32)

    def inclusive_scan(a):
        c = a
        d = 1
        while d < _T:
            c = c + jnp.concatenate(
                [jnp.zeros((d, _E), jnp.int32), c[:-d]], axis=0)
            d *= 2
        return c

    cs = inclusive_scan(a0i + a1i)
    counts = cs[_T - 1:_T, :]
    aligned = jnp.bitwise_and(counts + (_B - 1), ~(_B - 1))

    offs_cols = []
    run = jnp.zeros((1, 1), jnp.int32)
    for e in range(_E):
        offs_cols.append(run)
        run = run + aligned[:, e:e + 1]
    offs = jnp.concatenate(offs_cols, axis=1)
    total = run

    rank0 = jnp.sum(a0i * (cs - 1), axis=1, keepdims=True)
    rank1 = jnp.sum(a1i * (cs - 1), axis=1, keepdims=True)
    off0 = jnp.sum(a0i * offs, axis=1, keepdims=True)
    off1 = jnp.sum(a1i * offs, axis=1, keepdims=True)
    pos0_ref[...] = off0 + rank0
    pos1_ref[...] = off1 + rank1
    tw0_ref[...] = m1
    tw1_ref[...] = m2

    bstart = _B * jax.lax.broadcasted_iota(jnp.int32, (_NB, 1), 0)
    inb = jnp.logical_and(bstart >= offs, bstart < offs + aligned)
    e_row = jax.lax.broadcasted_iota(jnp.int32, (_NB, _E), 1)
    be = jnp.sum(jnp.where(inb, e_row, 0), axis=1, keepdims=True)
    bv = jnp.sum(inb.astype(jnp.int32), axis=1, keepdims=True)
    lastq = total - 1
    in_last = jnp.logical_and(lastq >= offs, lastq < offs + aligned)
    e_last = jnp.sum(jnp.where(
        in_last, jax.lax.broadcasted_iota(jnp.int32, (1, _E), 1), 0),
        axis=1, keepdims=True)
    be_ref[...] = jnp.where(bv > 0, be, e_last)
    bv_ref[...] = bv


def _ffn_kernel(be_ref, bv_ref, pos0_ref, pos1_ref, tw0_ref, tw1_ref,
                x_ref, w1_ref, w3_ref, w2_ref, out_ref,
                g_s, xg_s, ws_s, yacc_s):
    b = pl.program_id(0)
    i = pl.program_id(1)

    @pl.when(jnp.logical_and(b == 0, i == 0))
    def _init():
        out_ref[...] = jnp.zeros_like(out_ref)

    valid = bv_ref[b] > 0

    @pl.when(jnp.logical_and(valid, i == 0))
    def _prep():
        slots = _B * b + jax.lax.broadcasted_iota(jnp.int32, (_B, 1), 0)
        m0 = (pos0_ref[...] == slots)
        m1 = (pos1_ref[...] == slots)
        g = jnp.logical_or(m0, m1).astype(jnp.bfloat16)
        g_s[...] = g
        ws_s[...] = jnp.sum(jnp.where(m0, tw0_ref[...], 0.0) +
                            jnp.where(m1, tw1_ref[...], 0.0),
                            axis=1, keepdims=True)
        xg_s[...] = jax.lax.dot_general(
            g, x_ref[...], (((1,), (0,)), ((), ())),
            preferred_element_type=jnp.float32).astype(jnp.bfloat16)

    @pl.when(valid)
    def _body():
        xg = xg_s[...]
        w1 = w1_ref[0].astype(jnp.bfloat16)
        w3 = w3_ref[0].astype(jnp.bfloat16)
        w2 = w2_ref[0].astype(jnp.bfloat16)
        gp = jax.lax.dot_general(
            xg, w1, (((1,), (1,)), ((), ())),
            preferred_element_type=jnp.float32)
        up = jax.lax.dot_general(
            xg, w3, (((1,), (1,)), ((), ())),
            preferred_element_type=jnp.float32)
        h = (gp * jax.nn.sigmoid(gp) * up).astype(jnp.bfloat16)
        yh = jax.lax.dot_general(
            h, w2, (((1,), (1,)), ((), ())),
            preferred_element_type=jnp.float32)

        @pl.when(i == 0)
        def _first():
            yacc_s[...] = yh

        @pl.when(i == 1)
        def _last():
            y = yacc_s[...] + yh
            yw = (y * ws_s[...]).astype(jnp.bfloat16)
            out_ref[...] += jax.lax.dot_general(
                g_s[...], yw, (((0,), (0,)), ((), ())),
                preferred_element_type=jnp.float32)


@jax.jit
def kernel(hidden_states, gate_w, w1, w3, w2):
    f32 = jnp.float32
    router_out = pl.pallas_call(
        _router_kernel,
        out_shape=[
            jax.ShapeDtypeStruct((_T, 1), jnp.int32),
            jax.ShapeDtypeStruct((_T, 1), jnp.int32),
            jax.ShapeDtypeStruct((_T, 1), f32),
            jax.ShapeDtypeStruct((_T, 1), f32),
            jax.ShapeDtypeStruct((_NB, 1), jnp.int32),
            jax.ShapeDtypeStruct((_NB, 1), jnp.int32),
            jax.ShapeDtypeStruct((_T, _H), jnp.bfloat16),
        ],
    )(hidden_states, gate_w)
    pos0, pos1, tw0, tw1, be, bv, x_bf = router_out
    pos0 = pos0.reshape(1, _T)
    pos1 = pos1.reshape(1, _T)
    tw0 = tw0.reshape(1, _T)
    tw1 = tw1.reshape(1, _T)
    be = be.reshape(_NB)
    bv = bv.reshape(_NB)

    bf16 = jnp.bfloat16
    i2 = _I // 2
    grid_spec = pltpu.PrefetchScalarGridSpec(
        num_scalar_prefetch=2,
        grid=(_NB, 2),
        in_specs=[
            pl.BlockSpec((1, _T), lambda b, i, be, bv: (0, 0)),
            pl.BlockSpec((1, _T), lambda b, i, be, bv: (0, 0)),
            pl.BlockSpec((1, _T), lambda b, i, be, bv: (0, 0)),
            pl.BlockSpec((1, _T), lambda b, i, be, bv: (0, 0)),
            pl.BlockSpec((_T, _H), lambda b, i, be, bv: (0, 0)),
            pl.BlockSpec((1, i2, _H), lambda b, i, be, bv: (be[b], i, 0)),
            pl.BlockSpec((1, i2, _H), lambda b, i, be, bv: (be[b], i, 0)),
            pl.BlockSpec((1, _H, i2), lambda b, i, be, bv: (be[b], 0, i)),
        ],
        out_specs=pl.BlockSpec((_T, _H), lambda b, i, be, bv: (0, 0)),
        scratch_shapes=[
            pltpu.VMEM((_B, _T), bf16),
            pltpu.VMEM((_B, _H), bf16),
            pltpu.VMEM((_B, 1), f32),
            pltpu.VMEM((_B, _H), f32),
        ],
    )
    out = pl.pallas_call(
        _ffn_kernel,
        grid_spec=grid_spec,
        out_shape=jax.ShapeDtypeStruct((_T, _H), f32),
        compiler_params=pltpu.CompilerParams(
            dimension_semantics=("arbitrary", "arbitrary"),
            vmem_limit_bytes=64 * 1024 * 1024),
    )(be, bv, pos0, pos1, tw0, tw1, x_bf, w1, w3, w2)
    return out


# speedup vs baseline: 1.2646x; 1.0980x over previous
import functools

import jax
import jax.numpy as jnp
from jax.experimental import pallas as pl
import jax.experimental.pallas.tpu as pltpu

_T = 2048
_H = 1024
_I = 2816
_E = 8
_B = 256
_NB = 24


def _router_kernel(x_ref, gw_ref, pos0_ref, pos1_ref, tw0_ref, tw1_ref,
                   be_ref, bv_ref, xbf_ref):
    x = x_ref[...]
    xbf_ref[...] = x.astype(jnp.bfloat16)
    gw = gw_ref[...]
    logits = jax.lax.dot_general(
        x, gw, (((1,), (1,)), ((), ())),
        precision=jax.lax.Precision.DEFAULT,
        preferred_element_type=jnp.float32)
    m = jnp.max(logits, axis=1, keepdims=True)
    ex = jnp.exp(logits - m)
    probs = ex / jnp.sum(ex, axis=1, keepdims=True)

    iota_e = jax.lax.broadcasted_iota(jnp.int32, (_T, _E), 1)
    m1 = jnp.max(probs, axis=1, keepdims=True)
    i0 = jnp.min(jnp.where(probs == m1, iota_e, _E), axis=1, keepdims=True)
    a0 = (iota_e == i0)
    probs2 = jnp.where(a0, -1.0, probs)
    m2 = jnp.max(probs2, axis=1, keepdims=True)
    i1 = jnp.min(jnp.where(probs2 == m2, iota_e, _E), axis=1, keepdims=True)
    a1 = (iota_e == i1)

    a0i = a0.astype(jnp.int32)
    a1i = a1.astype(jnp.int32)

    def inclusive_scan(a):
        c = a
        d = 1
        while d < _T:
            c = c + jnp.concatenate(
                [jnp.zeros((d, _E), jnp.int32), c[:-d]], axis=0)
            d *= 2
        return c

    cs = inclusive_scan(a0i + a1i)
    counts = cs[_T - 1:_T, :]
    aligned = jnp.bitwise_and(counts + (_B - 1), ~(_B - 1))

    offs_cols = []
    run = jnp.zeros((1, 1), jnp.int32)
    for e in range(_E):
        offs_cols.append(run)
        run = run + aligned[:, e:e + 1]
    offs = jnp.concatenate(offs_cols, axis=1)
    total = run

    rank0 = jnp.sum(a0i * (cs - 1), axis=1, keepdims=True)
    rank1 = jnp.sum(a1i * (cs - 1), axis=1, keepdims=True)
    off0 = jnp.sum(a0i * offs, axis=1, keepdims=True)
    off1 = jnp.sum(a1i * offs, axis=1, keepdims=True)
    pos0_ref[...] = off0 + rank0
    pos1_ref[...] = off1 + rank1
    tw0_ref[...] = m1
    tw1_ref[...] = m2

    bstart = _B * jax.lax.broadcasted_iota(jnp.int32, (_NB, 1), 0)
    inb = jnp.logical_and(bstart >= offs, bstart < offs + aligned)
    e_row = jax.lax.broadcasted_iota(jnp.int32, (_NB, _E), 1)
    be = jnp.sum(jnp.where(inb, e_row, 0), axis=1, keepdims=True)
    bv = jnp.sum(inb.astype(jnp.int32), axis=1, keepdims=True)
    lastq = total - 1
    in_last = jnp.logical_and(lastq >= offs, lastq < offs + aligned)
    e_last = jnp.sum(jnp.where(
        in_last, jax.lax.broadcasted_iota(jnp.int32, (1, _E), 1), 0),
        axis=1, keepdims=True)
    be_ref[...] = jnp.where(bv > 0, be, e_last)
    bv_ref[...] = bv


def _ffn_kernel(be_ref, bv_ref, pos0_ref, pos1_ref, tw0_ref, tw1_ref,
                x_ref, w1_ref, w3_ref, w2_ref, out_ref, xg_all):
    i = pl.program_id(0)
    b = pl.program_id(1)

    @pl.when(jnp.logical_and(b == 0, i == 0))
    def _init():
        out_ref[...] = jnp.zeros_like(out_ref)

    @pl.when(bv_ref[b] > 0)
    def _body():
        slots = _B * b + jax.lax.broadcasted_iota(jnp.int32, (_B, 1), 0)
        m0 = (pos0_ref[...] == slots)
        m1 = (pos1_ref[...] == slots)
        g = jnp.logical_or(m0, m1).astype(jnp.bfloat16)
        wslot = jnp.sum(jnp.where(m0, tw0_ref[...], 0.0) +
                        jnp.where(m1, tw1_ref[...], 0.0),
                        axis=1, keepdims=True)

        @pl.when(i == 0)
        def _gather():
            xg_all[pl.ds(_B * b, _B), :] = jax.lax.dot_general(
                g, x_ref[...], (((1,), (0,)), ((), ())),
                preferred_element_type=jnp.float32).astype(jnp.bfloat16)

        xg = xg_all[pl.ds(_B * b, _B), :]
        w1 = w1_ref[0].astype(jnp.bfloat16)
        w3 = w3_ref[0].astype(jnp.bfloat16)
        w2 = w2_ref[0].astype(jnp.bfloat16)
        gp = jax.lax.dot_general(
            xg, w1, (((1,), (1,)), ((), ())),
            preferred_element_type=jnp.float32)
        up = jax.lax.dot_general(
            xg, w3, (((1,), (1,)), ((), ())),
            preferred_element_type=jnp.float32)
        h = (gp * jax.nn.sigmoid(gp) * up).astype(jnp.bfloat16)
        yh = jax.lax.dot_general(
            h, w2, (((1,), (1,)), ((), ())),
            preferred_element_type=jnp.float32)
        yw = (yh * wslot).astype(jnp.bfloat16)
        out_ref[...] += jax.lax.dot_general(
            g, yw, (((0,), (0,)), ((), ())),
            preferred_element_type=jnp.float32)


@jax.jit
def kernel(hidden_states, gate_w, w1, w3, w2):
    f32 = jnp.float32
    router_out = pl.pallas_call(
        _router_kernel,
        out_shape=[
            jax.ShapeDtypeStruct((_T, 1), jnp.int32),
            jax.ShapeDtypeStruct((_T, 1), jnp.int32),
            jax.ShapeDtypeStruct((_T, 1), f32),
            jax.ShapeDtypeStruct((_T, 1), f32),
            jax.ShapeDtypeStruct((_NB, 1), jnp.int32),
            jax.ShapeDtypeStruct((_NB, 1), jnp.int32),
            jax.ShapeDtypeStruct((_T, _H), jnp.bfloat16),
        ],
    )(hidden_states, gate_w)
    pos0, pos1, tw0, tw1, be, bv, x_bf = router_out
    pos0 = pos0.reshape(1, _T)
    pos1 = pos1.reshape(1, _T)
    tw0 = tw0.reshape(1, _T)
    tw1 = tw1.reshape(1, _T)
    be = be.reshape(_NB)
    bv = bv.reshape(_NB)

    bf16 = jnp.bfloat16
    i2 = _I // 2
    grid_spec = pltpu.PrefetchScalarGridSpec(
        num_scalar_prefetch=2,
        grid=(2, _NB),
        in_specs=[
            pl.BlockSpec((1, _T), lambda i, b, be, bv: (0, 0)),
            pl.BlockSpec((1, _T), lambda i, b, be, bv: (0, 0)),
            pl.BlockSpec((1, _T), lambda i, b, be, bv: (0, 0)),
            pl.BlockSpec((1, _T), lambda i, b, be, bv: (0, 0)),
            pl.BlockSpec((_T, _H), lambda i, b, be, bv: (0, 0)),
            pl.BlockSpec((1, i2, _H), lambda i, b, be, bv: (be[b], i, 0)),
            pl.BlockSpec((1, i2, _H), lambda i, b, be, bv: (be[b], i, 0)),
            pl.BlockSpec((1, _H, i2), lambda i, b, be, bv: (be[b], 0, i)),
        ],
        out_specs=pl.BlockSpec((_T, _H), lambda i, b, be, bv: (0, 0)),
        scratch_shapes=[
            pltpu.VMEM((_NB * _B, _H), bf16),
        ],
    )
    out = pl.pallas_call(
        _ffn_kernel,
        grid_spec=grid_spec,
        out_shape=jax.ShapeDtypeStruct((_T, _H), f32),
        compiler_params=pltpu.CompilerParams(
            dimension_semantics=("arbitrary", "arbitrary"),
            vmem_limit_bytes=64 * 1024 * 1024),
    )(be, bv, pos0, pos1, tw0, tw1, x_bf, w1, w3, w2)
    return out
